# baseline (speedup 1.0000x reference)
import functools

import jax
import jax.numpy as jnp
from jax import lax
from jax.experimental import pallas as pl
from jax.experimental.pallas import tpu as pltpu
from jax.experimental.pallas import tpu_sc as plsc

OUT_D = 64
NB_ACT = 26
BATCH = 16384
WIDTH = OUT_D * NB_ACT

SPLIT = 7680

NC = 2
NS = 16
L = 16
NW = NC * NS
B_PER_W = SPLIT // NW
G = 16
NG = B_PER_W // G
NBUF = 4

TC_BLK = 512
TC_ROWS = BATCH - SPLIT


def _sc_kernel(full_hbm, idx_hbm, out_hbm, idx_v, land_v, out_v,
               in_sems, out_sems):
    wid = lax.axis_index("s") * NC + lax.axis_index("c")
    base = wid * B_PER_W

    pltpu.sync_copy(idx_hbm.at[pl.ds(base, B_PER_W)], idx_v)

    lane = lax.iota(jnp.int32, L)

    def start_group(g, buf):
        j16 = lax.div(idx_v[pl.ds(g * G, L)], 2) * 128
        for l in range(L):
            col = pl.multiple_of(j16[l], 128)
            pltpu.make_async_copy(
                full_hbm.at[pl.ds(base + g * G + (l & ~7), 8),
                            pl.ds(col, 128)],
                land_v.at[buf, l],
                in_sems.at[buf],
            ).start()

    def wait_group(buf):
        for l in range(L):
            pltpu.make_async_copy(
                full_hbm.at[pl.ds(0, 8), pl.ds(0, 128)],
                land_v.at[buf, l],
                in_sems.at[buf],
            ).wait()

    def out_copy(g, buf):
        return pltpu.make_async_copy(
            out_v.at[buf],
            out_hbm.at[pl.ds(base + g * G, G)],
            out_sems.at[buf],
        )

    for b in range(NBUF):
        start_group(b, b)

    sub = lax.rem(lane, 8)

    def body(g, _):
        buf = lax.rem(g, NBUF)
        wait_group(buf)

        @pl.when(g >= NBUF)
        def _():
            out_copy(g - NBUF, buf).wait()

        v16 = idx_v[pl.ds(g * G, L)]
        h16 = lax.rem(v16, 2) * OUT_D
        bufl = jnp.full((L,), buf, jnp.int32)
        for c in range(OUT_D):
            vals = plsc.load_gather(land_v, [bufl, lane, sub, h16 + c])
            plsc.store_scatter(
                out_v, [bufl, lane, jnp.full((L,), c, jnp.int32)], vals
            )
        out_copy(g, buf).start()

        @pl.when(g + NBUF < NG)
        def _():
            start_group(g + NBUF, buf)

        return 0

    lax.fori_loop(0, NG, body, 0)

    for b in range(NBUF):
        out_copy(NG - NBUF + b, lax.rem(NG - NBUF + b, NBUF)).wait()


def _tc_kernel(x_ref, idx_ref, out_ref):
    idx = idx_ref[...]
    x = x_ref[...]
    idx2 = lax.div(idx, 2)
    lvl = [x[:, j * 128:(j + 1) * 128] for j in range(NB_ACT // 2)]
    lvl = lvl + [lvl[0]] * (16 - len(lvl))
    for k in range(4):
        bit = lax.rem(lax.shift_right_logical(idx2, k), 2) == 1
        lvl = [jnp.where(bit, lvl[2 * i + 1], lvl[2 * i])
               for i in range(len(lvl) // 2)]
    acc = lvl[0]
    out_ref[...] = jnp.where(
        lax.rem(idx, 2) == 0, acc[:, :OUT_D], acc[:, OUT_D:]
    )


@jax.jit
def _run(full_output, idx1d, idx2d):
    mesh = plsc.VectorSubcoreMesh(core_axis_name="c", subcore_axis_name="s")
    sc_out = pl.kernel(
        _sc_kernel,
        out_type=jax.ShapeDtypeStruct((SPLIT, OUT_D), jnp.float32),
        mesh=mesh,
        scratch_types=[
            pltpu.VMEM((B_PER_W,), jnp.int32),
            pltpu.VMEM((NBUF, G, 8, 128), jnp.float32),
            pltpu.VMEM((NBUF, G, OUT_D), jnp.float32),
            pltpu.SemaphoreType.DMA((NBUF,)),
            pltpu.SemaphoreType.DMA((NBUF,)),
        ],
        compiler_params=pltpu.CompilerParams(needs_layout_passes=False),
    )(full_output, idx1d)

    nblk = TC_ROWS // TC_BLK
    blk0 = SPLIT // TC_BLK
    tc_out = pl.pallas_call(
        _tc_kernel,
        grid=(nblk,),
        in_specs=[
            pl.BlockSpec((TC_BLK, WIDTH), lambda i: (blk0 + i, 0)),
            pl.BlockSpec((TC_BLK, 1), lambda i: (blk0 + i, 0)),
        ],
        out_specs=pl.BlockSpec((TC_BLK, OUT_D), lambda i: (i, 0)),
        out_shape=jax.ShapeDtypeStruct((TC_ROWS, OUT_D), jnp.float32),
    )(full_output, idx2d)

    return jnp.concatenate([sc_out, tc_out], axis=0)


def kernel(full_output, indices):
    idx2d = indices.astype(jnp.int32)
    idx1d = idx2d.reshape(BATCH)
    return _run(full_output, idx1d, idx2d)


# speedup vs baseline: 1.0233x; 1.0233x over previous
import functools

import jax
import jax.numpy as jnp
from jax import lax
from jax.experimental import pallas as pl
from jax.experimental.pallas import tpu as pltpu
from jax.experimental.pallas import tpu_sc as plsc

OUT_D = 64
NB_ACT = 26
BATCH = 16384
WIDTH = OUT_D * NB_ACT

SPLIT = 8704

NC = 2
NS = 16
L = 16
NW = NC * NS
B_PER_W = SPLIT // NW
G = 16
NG = B_PER_W // G
NBUF = 4

TC_BLK = 512
TC_ROWS = BATCH - SPLIT


def _sc_kernel(full_hbm, idx_hbm, out_hbm, idx_v, land_v, out_v,
               in_sems, out_sems):
    wid = lax.axis_index("s") * NC + lax.axis_index("c")
    base = wid * B_PER_W

    pltpu.sync_copy(idx_hbm.at[pl.ds(base, B_PER_W)], idx_v)

    lane = lax.iota(jnp.int32, L)

    def start_group(g, buf):
        j16 = lax.div(idx_v[pl.ds(g * G, L)], 2) * 128
        for l in range(L):
            col = pl.multiple_of(j16[l], 128)
            pltpu.make_async_copy(
                full_hbm.at[pl.ds(base + g * G + (l & ~7), 8),
                            pl.ds(col, 128)],
                land_v.at[buf, l],
                in_sems.at[buf],
            ).start()

    def wait_group(buf):
        for l in range(L):
            pltpu.make_async_copy(
                full_hbm.at[pl.ds(0, 8), pl.ds(0, 128)],
                land_v.at[buf, l],
                in_sems.at[buf],
            ).wait()

    def out_copy(g, buf):
        return pltpu.make_async_copy(
            out_v.at[buf],
            out_hbm.at[pl.ds(base + g * G, G)],
            out_sems.at[buf],
        )

    for b in range(NBUF):
        start_group(b, b)

    sub = lax.rem(lane, 8)

    def body(g, _):
        buf = lax.rem(g, NBUF)
        wait_group(buf)

        @pl.when(g >= NBUF)
        def _():
            out_copy(g - NBUF, buf).wait()

        v16 = idx_v[pl.ds(g * G, L)]
        h16 = lax.rem(v16, 2) * OUT_D
        bufl = jnp.full((L,), buf, jnp.int32)
        for c in range(OUT_D):
            vals = plsc.load_gather(land_v, [bufl, lane, sub, h16 + c])
            plsc.store_scatter(
                out_v, [bufl, lane, jnp.full((L,), c, jnp.int32)], vals
            )
        out_copy(g, buf).start()

        @pl.when(g + NBUF < NG)
        def _():
            start_group(g + NBUF, buf)

        return 0

    lax.fori_loop(0, NG, body, 0)

    for b in range(NBUF):
        out_copy(NG - NBUF + b, lax.rem(NG - NBUF + b, NBUF)).wait()


def _tc_kernel(x_ref, idx_ref, out_ref):
    idx = idx_ref[...]
    x = x_ref[...]
    idx2 = lax.div(idx, 2)
    lvl = [x[:, j * 128:(j + 1) * 128] for j in range(NB_ACT // 2)]
    lvl = lvl + [lvl[0]] * (16 - len(lvl))
    for k in range(4):
        bit = lax.rem(lax.shift_right_logical(idx2, k), 2) == 1
        lvl = [jnp.where(bit, lvl[2 * i + 1], lvl[2 * i])
               for i in range(len(lvl) // 2)]
    acc = lvl[0]
    out_ref[...] = jnp.where(
        lax.rem(idx, 2) == 0, acc[:, :OUT_D], acc[:, OUT_D:]
    )


@jax.jit
def _run(full_output, idx1d, idx2d):
    mesh = plsc.VectorSubcoreMesh(core_axis_name="c", subcore_axis_name="s")
    sc_out = pl.kernel(
        _sc_kernel,
        out_type=jax.ShapeDtypeStruct((SPLIT, OUT_D), jnp.float32),
        mesh=mesh,
        scratch_types=[
            pltpu.VMEM((B_PER_W,), jnp.int32),
            pltpu.VMEM((NBUF, G, 8, 128), jnp.float32),
            pltpu.VMEM((NBUF, G, OUT_D), jnp.float32),
            pltpu.SemaphoreType.DMA((NBUF,)),
            pltpu.SemaphoreType.DMA((NBUF,)),
        ],
        compiler_params=pltpu.CompilerParams(needs_layout_passes=False),
    )(full_output, idx1d)

    nblk = TC_ROWS // TC_BLK
    blk0 = SPLIT // TC_BLK
    tc_out = pl.pallas_call(
        _tc_kernel,
        grid=(nblk,),
        in_specs=[
            pl.BlockSpec((TC_BLK, WIDTH), lambda i: (blk0 + i, 0)),
            pl.BlockSpec((TC_BLK, 1), lambda i: (blk0 + i, 0)),
        ],
        out_specs=pl.BlockSpec((TC_BLK, OUT_D), lambda i: (i, 0)),
        out_shape=jax.ShapeDtypeStruct((TC_ROWS, OUT_D), jnp.float32),
    )(full_output, idx2d)

    return jnp.concatenate([sc_out, tc_out], axis=0)


def kernel(full_output, indices):
    idx2d = indices.astype(jnp.int32)
    idx1d = idx2d.reshape(BATCH)
    return _run(full_output, idx1d, idx2d)


# speedup vs baseline: 1.0395x; 1.0158x over previous
import functools

import jax
import jax.numpy as jnp
from jax import lax
from jax.experimental import pallas as pl
from jax.experimental.pallas import tpu as pltpu
from jax.experimental.pallas import tpu_sc as plsc

OUT_D = 64
NB_ACT = 26
BATCH = 16384
WIDTH = OUT_D * NB_ACT

SPLIT = 9216

NC = 2
NS = 16
L = 16
NW = NC * NS
B_PER_W = SPLIT // NW
G = 16
NG = B_PER_W // G
NBUF = 4

TC_BLK = 512
TC_ROWS = BATCH - SPLIT


def _sc_kernel(full_hbm, idx_hbm, out_hbm, idx_v, land_v, out_v,
               in_sems, out_sems):
    wid = lax.axis_index("s") * NC + lax.axis_index("c")
    base = wid * B_PER_W

    pltpu.sync_copy(idx_hbm.at[pl.ds(base, B_PER_W)], idx_v)

    lane = lax.iota(jnp.int32, L)

    def start_group(g, buf):
        j16 = lax.div(idx_v[pl.ds(g * G, L)], 2) * 128
        for l in range(L):
            col = pl.multiple_of(j16[l], 128)
            pltpu.make_async_copy(
                full_hbm.at[pl.ds(base + g * G + (l & ~7), 8),
                            pl.ds(col, 128)],
                land_v.at[buf, l],
                in_sems.at[buf],
            ).start()

    def wait_group(buf):
        for l in range(L):
            pltpu.make_async_copy(
                full_hbm.at[pl.ds(0, 8), pl.ds(0, 128)],
                land_v.at[buf, l],
                in_sems.at[buf],
            ).wait()

    def out_copy(g, buf):
        return pltpu.make_async_copy(
            out_v.at[buf],
            out_hbm.at[pl.ds(base + g * G, G)],
            out_sems.at[buf],
        )

    for b in range(NBUF):
        start_group(b, b)

    sub = lax.rem(lane, 8)

    def body(g, _):
        buf = lax.rem(g, NBUF)
        wait_group(buf)

        @pl.when(g >= NBUF)
        def _():
            out_copy(g - NBUF, buf).wait()

        v16 = idx_v[pl.ds(g * G, L)]
        h16 = lax.rem(v16, 2) * OUT_D
        bufl = jnp.full((L,), buf, jnp.int32)
        for c in range(OUT_D):
            vals = plsc.load_gather(land_v, [bufl, lane, sub, h16 + c])
            plsc.store_scatter(
                out_v, [bufl, lane, jnp.full((L,), c, jnp.int32)], vals
            )
        out_copy(g, buf).start()

        @pl.when(g + NBUF < NG)
        def _():
            start_group(g + NBUF, buf)

        return 0

    lax.fori_loop(0, NG, body, 0)

    for b in range(NBUF):
        out_copy(NG - NBUF + b, lax.rem(NG - NBUF + b, NBUF)).wait()


def _tc_kernel(x_ref, idx_ref, out_ref):
    idx = idx_ref[...]
    x = x_ref[...]
    idx2 = lax.div(idx, 2)
    lvl = [x[:, j * 128:(j + 1) * 128] for j in range(NB_ACT // 2)]
    lvl = lvl + [lvl[0]] * (16 - len(lvl))
    for k in range(4):
        bit = lax.rem(lax.shift_right_logical(idx2, k), 2) == 1
        lvl = [jnp.where(bit, lvl[2 * i + 1], lvl[2 * i])
               for i in range(len(lvl) // 2)]
    acc = lvl[0]
    out_ref[...] = jnp.where(
        lax.rem(idx, 2) == 0, acc[:, :OUT_D], acc[:, OUT_D:]
    )


@jax.jit
def _run(full_output, idx1d, idx2d):
    mesh = plsc.VectorSubcoreMesh(core_axis_name="c", subcore_axis_name="s")
    sc_out = pl.kernel(
        _sc_kernel,
        out_type=jax.ShapeDtypeStruct((SPLIT, OUT_D), jnp.float32),
        mesh=mesh,
        scratch_types=[
            pltpu.VMEM((B_PER_W,), jnp.int32),
            pltpu.VMEM((NBUF, G, 8, 128), jnp.float32),
            pltpu.VMEM((NBUF, G, OUT_D), jnp.float32),
            pltpu.SemaphoreType.DMA((NBUF,)),
            pltpu.SemaphoreType.DMA((NBUF,)),
        ],
        compiler_params=pltpu.CompilerParams(needs_layout_passes=False),
    )(full_output, idx1d)

    nblk = TC_ROWS // TC_BLK
    blk0 = SPLIT // TC_BLK
    tc_out = pl.pallas_call(
        _tc_kernel,
        grid=(nblk,),
        in_specs=[
            pl.BlockSpec((TC_BLK, WIDTH), lambda i: (blk0 + i, 0)),
            pl.BlockSpec((TC_BLK, 1), lambda i: (blk0 + i, 0)),
        ],
        out_specs=pl.BlockSpec((TC_BLK, OUT_D), lambda i: (i, 0)),
        out_shape=jax.ShapeDtypeStruct((TC_ROWS, OUT_D), jnp.float32),
    )(full_output, idx2d)

    return jnp.concatenate([sc_out, tc_out], axis=0)


def kernel(full_output, indices):
    idx2d = indices.astype(jnp.int32)
    idx1d = idx2d.reshape(BATCH)
    return _run(full_output, idx1d, idx2d)


# speedup vs baseline: 1.0398x; 1.0003x over previous
import jax
import jax.numpy as jnp
from jax import lax
from jax.experimental import pallas as pl
from jax.experimental.pallas import tpu as pltpu
from jax.experimental.pallas import tpu_sc as plsc

OUT_D = 64
NB_ACT = 26
BATCH = 16384
WIDTH = OUT_D * NB_ACT

SPLIT = 9216

NC = 2
NS = 16
L = 16
NW = NC * NS
B_PER_W = SPLIT // NW
G = 16
NG = B_PER_W // G
NBUF = 4

TC_BLK = 512
TC_ROWS = BATCH - SPLIT


def _sc_kernel(full_hbm, idx_hbm, out_hbm, idx_v, land_v, out_v,
               in_sems, out_sems):
    wid = lax.axis_index("s") * NC + lax.axis_index("c")
    base = wid * B_PER_W

    pltpu.sync_copy(idx_hbm.at[pl.ds(base, B_PER_W)], idx_v)

    lane = lax.iota(jnp.int32, L)

    def start_group(g, buf):
        j16 = lax.div(idx_v[pl.ds(g * G, L)], 2) * 128
        for l in range(L):
            col = pl.multiple_of(j16[l], 128)
            pltpu.make_async_copy(
                full_hbm.at[pl.ds(base + g * G + (l & ~7), 8),
                            pl.ds(col, 128)],
                land_v.at[buf, l],
                in_sems.at[buf],
            ).start()

    def wait_group(buf):
        for l in range(L):
            pltpu.make_async_copy(
                full_hbm.at[pl.ds(0, 8), pl.ds(0, 128)],
                land_v.at[buf, l],
                in_sems.at[buf],
            ).wait()

    def out_copy(g, buf):
        return pltpu.make_async_copy(
            out_v.at[buf],
            out_hbm.at[pl.ds(base + g * G, G)],
            out_sems.at[buf],
        )

    for b in range(NBUF):
        start_group(b, b)

    sub = lax.rem(lane, 8)

    def body(g, _):
        buf = lax.rem(g, NBUF)
        wait_group(buf)

        @pl.when(g >= NBUF)
        def _():
            out_copy(g - NBUF, buf).wait()

        v16 = idx_v[pl.ds(g * G, L)]
        h16 = lax.rem(v16, 2) * OUT_D
        bufl = jnp.full((L,), buf, jnp.int32)
        for c in range(OUT_D):
            vals = plsc.load_gather(land_v, [bufl, lane, sub, h16 + c])
            plsc.store_scatter(
                out_v, [bufl, lane, jnp.full((L,), c, jnp.int32)], vals
            )
        out_copy(g, buf).start()

        @pl.when(g + NBUF < NG)
        def _():
            start_group(g + NBUF, buf)

        return 0

    lax.fori_loop(0, NG, body, 0)

    for b in range(NBUF):
        out_copy(NG - NBUF + b, lax.rem(NG - NBUF + b, NBUF)).wait()


def _tc_kernel(x_ref, idx_ref, out_ref):
    idx = idx_ref[...]
    x = x_ref[...]
    idx2 = lax.div(idx, 2)
    lvl = [x[:, j * 128:(j + 1) * 128] for j in range(NB_ACT // 2)]
    lvl = lvl + [lvl[0]] * (16 - len(lvl))
    for k in range(4):
        bit = lax.rem(lax.shift_right_logical(idx2, k), 2) == 1
        lvl = [jnp.where(bit, lvl[2 * i + 1], lvl[2 * i])
               for i in range(len(lvl) // 2)]
    acc = lvl[0]
    out_ref[...] = jnp.where(
        lax.rem(idx, 2) == 0, acc[:, :OUT_D], acc[:, OUT_D:]
    )


@jax.jit
def _run(full_output, idx1d, idx2d):
    mesh = plsc.VectorSubcoreMesh(core_axis_name="c", subcore_axis_name="s")
    sc_out = pl.kernel(
        _sc_kernel,
        out_type=jax.ShapeDtypeStruct((SPLIT, OUT_D), jnp.float32),
        mesh=mesh,
        scratch_types=[
            pltpu.VMEM((B_PER_W,), jnp.int32),
            pltpu.VMEM((NBUF, G, 8, 128), jnp.float32),
            pltpu.VMEM((NBUF, G, OUT_D), jnp.float32),
            pltpu.SemaphoreType.DMA((NBUF,)),
            pltpu.SemaphoreType.DMA((NBUF,)),
        ],
        compiler_params=pltpu.CompilerParams(needs_layout_passes=False),
    )(full_output, idx1d)

    nblk = TC_ROWS // TC_BLK
    blk0 = SPLIT // TC_BLK
    tc_out = pl.pallas_call(
        _tc_kernel,
        grid=(nblk,),
        in_specs=[
            pl.BlockSpec((TC_BLK, WIDTH), lambda i: (blk0 + i, 0)),
            pl.BlockSpec((TC_BLK, 1), lambda i: (blk0 + i, 0)),
        ],
        out_specs=pl.BlockSpec((TC_BLK, OUT_D), lambda i: (i, 0)),
        out_shape=jax.ShapeDtypeStruct((TC_ROWS, OUT_D), jnp.float32),
    )(full_output, idx2d)

    return jnp.concatenate([sc_out, tc_out], axis=0)


def kernel(full_output, indices):
    idx2d = indices.astype(jnp.int32)
    idx1d = idx2d.reshape(BATCH)
    return _run(full_output, idx1d, idx2d)
